# Initial kernel scaffold; baseline (speedup 1.0000x reference)
import jax
import jax.numpy as jnp
from jax.experimental import pallas as pl


def kernel(boxes, scores):
    raise NotImplementedError("write your pallas kernel here")


# speedup vs baseline: 12.6319x; 12.6319x over previous
import functools

import jax
import jax.numpy as jnp
from jax import lax
from jax.experimental import pallas as pl
from jax.experimental.pallas import tpu as pltpu
from jax.experimental.pallas import tpu_sc as plsc

_N = 1000
_P = 1024
_L = 16
_NCHUNK = _P // _L
_SIGMA = 0.5
_THR = 0.05
_BIG_I32 = 2**31 - 1


def _snms_body(hx1, hy1, hx2, hy2, hm, out, vx1, vy1, vx2, vy2, var, vm, vfin):
    @pl.when((lax.axis_index("c") == 0) & (lax.axis_index("s") == 0))
    def _():
        pltpu.sync_copy(hx1, vx1)
        pltpu.sync_copy(hy1, vy1)
        pltpu.sync_copy(hx2, vx2)
        pltpu.sync_copy(hy2, vy2)
        pltpu.sync_copy(hm, vm)

        lanes = lax.iota(jnp.int32, _L)

        dnums = lax.GatherDimensionNumbers(
            offset_dims=(), collapsed_slice_dims=(0,), start_index_map=(0,))

        def perm(x, idx):
            return lax.gather(x, idx[:, None], dnums, (1,),
                              mode=lax.GatherScatterMode.PROMISE_IN_BOUNDS)

        def bcast_max(x):
            for sh in (8, 4, 2, 1):
                x = jnp.maximum(x, perm(x, lanes ^ sh))
            return x

        def bcast_min_i32(x):
            for sh in (8, 4, 2, 1):
                x = jnp.minimum(x, perm(x, lanes ^ sh))
            return x

        zeros = jnp.zeros((_L,), jnp.float32)
        for c in range(_NCHUNK):
            sl = pl.ds(c * _L, _L)
            var[sl] = (vx2[sl] - vx1[sl]) * (vy2[sl] - vy1[sl])
            vfin[sl] = zeros

        bv = jnp.full((_L,), -2.0, jnp.float32)
        bi = jnp.zeros((_L,), jnp.int32)
        for c in range(_NCHUNK):
            sl = pl.ds(c * _L, _L)
            mc = vm[sl]
            gt = mc > bv
            bv = jnp.where(gt, mc, bv)
            bi = jnp.where(gt, lanes + c * _L, bi)
        v = bcast_max(bv)
        bo = bcast_min_i32(jnp.where(bv == v, bi, _BIG_I32))

        lane0 = lanes == 0
        neg1 = jnp.full((_L,), -1.0, jnp.float32)

        def body(_, carry):
            bo, v = carry
            plsc.store_scatter(vfin, [bo], v, mask=lane0)
            plsc.store_scatter(vm, [bo], neg1, mask=lane0)
            bx1 = plsc.load_gather(vx1, [bo])
            by1 = plsc.load_gather(vy1, [bo])
            bx2 = plsc.load_gather(vx2, [bo])
            by2 = plsc.load_gather(vy2, [bo])
            a_i = plsc.load_gather(var, [bo])

            bv = jnp.full((_L,), -2.0, jnp.float32)
            bi = jnp.zeros((_L,), jnp.int32)
            for c in range(_NCHUNK):
                sl = pl.ds(c * _L, _L)
                xx1 = jnp.maximum(bx1, vx1[sl])
                yy1 = jnp.maximum(by1, vy1[sl])
                xx2 = jnp.minimum(bx2, vx2[sl])
                yy2 = jnp.minimum(by2, vy2[sl])
                inter = jnp.maximum(xx2 - xx1, 0.0) * jnp.maximum(yy2 - yy1, 0.0)
                iou = inter / (a_i + var[sl] - inter + 1e-7)
                dec = jnp.exp(iou * iou * (-1.0 / _SIGMA))
                mc = vm[sl]
                mn = jnp.where(mc >= 0.0, mc * dec, mc)
                vm[sl] = mn
                gt = mn > bv
                bv = jnp.where(gt, mn, bv)
                bi = jnp.where(gt, lanes + c * _L, bi)
            vv = bcast_max(bv)
            bo2 = bcast_min_i32(jnp.where(bv == vv, bi, _BIG_I32))
            return bo2, vv

        lax.fori_loop(0, _N, body, (bo, v))

        for c in range(_NCHUNK):
            sl = pl.ds(c * _L, _L)
            f = vfin[sl]
            vfin[sl] = jnp.where(f >= _THR, f, 0.0)
        pltpu.sync_copy(vfin, out)


_snms = functools.partial(
    pl.kernel,
    out_type=jax.ShapeDtypeStruct((_P,), jnp.float32),
    mesh=plsc.VectorSubcoreMesh(core_axis_name="c", subcore_axis_name="s",
                                num_cores=2, num_subcores=16),
    scratch_types=[pltpu.VMEM((_P,), jnp.float32) for _ in range(7)],
    compiler_params=pltpu.CompilerParams(needs_layout_passes=False),
)(_snms_body)


@jax.jit
def kernel(boxes, scores):
    pad = _P - _N
    return _snms(
        jnp.pad(boxes[:, 0], (0, pad)),
        jnp.pad(boxes[:, 1], (0, pad)),
        jnp.pad(boxes[:, 2], (0, pad)),
        jnp.pad(boxes[:, 3], (0, pad)),
        jnp.pad(scores, (0, pad), constant_values=-1.0),
    )[:_N]
